# baseline (speedup 1.0000x reference)
import functools

import jax
import jax.numpy as jnp
from jax import lax
from jax.experimental import pallas as pl
from jax.experimental.pallas import tpu as pltpu
from jax.experimental.pallas import tpu_sc as plsc

B = 4096
EMB = 32
LANES = 128
_QC = 4


def _make_sc_gather():
    info = plsc.get_sparse_core_info()
    nc, ns = info.num_cores, info.num_subcores
    nw = nc * ns
    b_per_w = B // nw
    rounds = b_per_w // 16

    mesh = plsc.VectorSubcoreMesh(core_axis_name="c", subcore_axis_name="s")

    @functools.partial(
        pl.kernel,
        mesh=mesh,
        compiler_params=pltpu.CompilerParams(needs_layout_passes=False),
        out_type=[
            jax.ShapeDtypeStruct((EMB, B), jnp.float32),
            jax.ShapeDtypeStruct((EMB, B), jnp.float32),
        ],
        scratch_types=[
            pltpu.VMEM((B,), jnp.int32),
            pltpu.VMEM((B,), jnp.int32),
            pltpu.VMEM((2 * _QC, EMB, LANES), jnp.float32),
            pltpu.VMEM((2 * _QC, EMB, LANES), jnp.float32),
            pltpu.VMEM((EMB, b_per_w), jnp.float32),
            pltpu.VMEM((EMB, b_per_w), jnp.float32),
            pltpu.SemaphoreType.DMA,
            pltpu.SemaphoreType.DMA,
            pltpu.SemaphoreType.DMA,
            pltpu.SemaphoreType.DMA,
        ],
    )
    def sc_gather(users_hbm, items_hbm, utabT_hbm, itabT_hbm,
                  uoutT_hbm, ioutT_hbm,
                  uidx_all, iidx_all, ublk, iblk, urows, irows,
                  usem0, usem1, isem0, isem1):
        usems, isems = (usem0, usem1), (isem0, isem1)
        wid = lax.axis_index("s") * nc + lax.axis_index("c")
        base = pl.multiple_of(wid * b_per_w, b_per_w)
        pltpu.sync_copy(users_hbm, uidx_all)
        pltpu.sync_copy(items_hbm, iidx_all)

        row16 = lax.iota(jnp.int32, 16)

        nq = 16 // _QC

        def round_body(r, _):
            start = pl.multiple_of(base + r * 16, 16)
            uvec = uidx_all[pl.ds(start, 16)]
            ivec = iidx_all[pl.ds(start, 16)]

            def fire(q):
                slot = (q % 2) * _QC
                copies = []
                for j in range(_QC):
                    u = uvec[q * _QC + j]
                    uoff = pl.multiple_of((u // LANES) * LANES, LANES)
                    copies.append(pltpu.async_copy(
                        utabT_hbm.at[:, pl.ds(uoff, LANES)],
                        ublk.at[slot + j], usems[q % 2]))
                    i = ivec[q * _QC + j]
                    ioff = pl.multiple_of((i // LANES) * LANES, LANES)
                    copies.append(pltpu.async_copy(
                        itabT_hbm.at[:, pl.ds(ioff, LANES)],
                        iblk.at[slot + j], isems[q % 2]))
                return copies

            def extract(q):
                slot = (q % 2) * _QC
                for j in range(_QC):
                    b16 = jnp.full((16,), r * 16 + q * _QC + j, dtype=jnp.int32)
                    j16 = jnp.full((16,), slot + j, dtype=jnp.int32)
                    for vec, blk, rows in ((uvec, ublk, urows),
                                           (ivec, iblk, irows)):
                        lane16 = jnp.full(
                            (16,), vec[q * _QC + j] % LANES, dtype=jnp.int32)
                        lo = plsc.load_gather(blk, [j16, row16, lane16])
                        hi = plsc.load_gather(blk, [j16, row16 + 16, lane16])
                        plsc.store_scatter(rows, [row16, b16], lo)
                        plsc.store_scatter(rows, [row16 + 16, b16], hi)

            inflight = fire(0)
            for q in range(nq):
                nxt = fire(q + 1) if q + 1 < nq else []
                for c in inflight:
                    c.wait()
                extract(q)
                inflight = nxt
            return ()

        lax.fori_loop(0, rounds, round_body, ())

        pltpu.sync_copy(urows, uoutT_hbm.at[:, pl.ds(base, b_per_w)])
        pltpu.sync_copy(irows, ioutT_hbm.at[:, pl.ds(base, b_per_w)])

    return sc_gather


_sc_gather = _make_sc_gather()


_BM = 1024


def _matmul_body(a_ref, b_ref, o_ref):
    o_ref[...] = lax.dot_general(
        a_ref[...], b_ref[...],
        (((0,), (0,)), ((), ())),
        preferred_element_type=jnp.float32,
    )


def _score_matmul(uT, iT):
    grid = (B // _BM,)
    return pl.pallas_call(
        _matmul_body,
        grid=grid,
        in_specs=[
            pl.BlockSpec((EMB, _BM), lambda i: (0, i)),
            pl.BlockSpec((EMB, B), lambda i: (0, 0)),
        ],
        out_specs=pl.BlockSpec((_BM, B), lambda i: (i, 0)),
        out_shape=jax.ShapeDtypeStruct((B, B), jnp.float32),
    )(uT, iT)


def kernel(users, pos_items, user_table, item_table):
    uT, iT = _sc_gather(users, pos_items, user_table.T, item_table.T)
    return _score_matmul(uT, iT)


# speedup vs baseline: 1.0055x; 1.0055x over previous
import functools

import jax
import jax.numpy as jnp
from jax import lax
from jax.experimental import pallas as pl
from jax.experimental.pallas import tpu as pltpu
from jax.experimental.pallas import tpu_sc as plsc

B = 4096
EMB = 32
LANES = 128
_QC = 4


def _make_sc_gather():
    info = plsc.get_sparse_core_info()
    nc, ns = info.num_cores, info.num_subcores
    nw = nc * ns
    b_per_w = B // nw
    rounds = b_per_w // 16

    mesh = plsc.VectorSubcoreMesh(core_axis_name="c", subcore_axis_name="s")

    @functools.partial(
        pl.kernel,
        mesh=mesh,
        compiler_params=pltpu.CompilerParams(needs_layout_passes=False),
        out_type=[
            jax.ShapeDtypeStruct((EMB, B), jnp.float32),
            jax.ShapeDtypeStruct((EMB, B), jnp.float32),
        ],
        scratch_types=[
            pltpu.VMEM((B,), jnp.int32),
            pltpu.VMEM((B,), jnp.int32),
            pltpu.VMEM((2 * _QC, EMB, LANES), jnp.float32),
            pltpu.VMEM((2 * _QC, EMB, LANES), jnp.float32),
            pltpu.VMEM((EMB, b_per_w), jnp.float32),
            pltpu.VMEM((EMB, b_per_w), jnp.float32),
            pltpu.SemaphoreType.DMA,
            pltpu.SemaphoreType.DMA,
            pltpu.SemaphoreType.DMA,
            pltpu.SemaphoreType.DMA,
        ],
    )
    def sc_gather(users_hbm, items_hbm, utabT_hbm, itabT_hbm,
                  uoutT_hbm, ioutT_hbm,
                  uidx_all, iidx_all, ublk, iblk, urows, irows,
                  usem0, usem1, isem0, isem1):
        usems, isems = (usem0, usem1), (isem0, isem1)
        wid = lax.axis_index("s") * nc + lax.axis_index("c")
        base = pl.multiple_of(wid * b_per_w, b_per_w)
        pltpu.sync_copy(users_hbm, uidx_all)
        pltpu.sync_copy(items_hbm, iidx_all)

        row16 = lax.iota(jnp.int32, 16)

        nq = 16 // _QC

        def round_body(r, _):
            start = pl.multiple_of(base + r * 16, 16)
            uvec = uidx_all[pl.ds(start, 16)]
            ivec = iidx_all[pl.ds(start, 16)]

            def fire(q):
                slot = (q % 2) * _QC
                copies = []
                for j in range(_QC):
                    u = uvec[q * _QC + j]
                    uoff = pl.multiple_of((u // LANES) * LANES, LANES)
                    copies.append(pltpu.async_copy(
                        utabT_hbm.at[:, pl.ds(uoff, LANES)],
                        ublk.at[slot + j], usems[q % 2]))
                    i = ivec[q * _QC + j]
                    ioff = pl.multiple_of((i // LANES) * LANES, LANES)
                    copies.append(pltpu.async_copy(
                        itabT_hbm.at[:, pl.ds(ioff, LANES)],
                        iblk.at[slot + j], isems[q % 2]))
                return copies

            def extract(q):
                slot = (q % 2) * _QC
                for j in range(_QC):
                    b16 = jnp.full((16,), r * 16 + q * _QC + j, dtype=jnp.int32)
                    j16 = jnp.full((16,), slot + j, dtype=jnp.int32)
                    for vec, blk, rows in ((uvec, ublk, urows),
                                           (ivec, iblk, irows)):
                        lane16 = jnp.full(
                            (16,), vec[q * _QC + j] % LANES, dtype=jnp.int32)
                        lo = plsc.load_gather(blk, [j16, row16, lane16])
                        hi = plsc.load_gather(blk, [j16, row16 + 16, lane16])
                        plsc.store_scatter(rows, [row16, b16], lo)
                        plsc.store_scatter(rows, [row16 + 16, b16], hi)

            inflight = fire(0)
            for q in range(nq):
                nxt = fire(q + 1) if q + 1 < nq else []
                for c in inflight:
                    c.wait()
                extract(q)
                inflight = nxt
            return ()

        lax.fori_loop(0, rounds, round_body, ())

        pltpu.sync_copy(urows, uoutT_hbm.at[:, pl.ds(base, b_per_w)])
        pltpu.sync_copy(irows, ioutT_hbm.at[:, pl.ds(base, b_per_w)])

    return sc_gather


_sc_gather = _make_sc_gather()


_BM = 256


def _matmul_body(a_ref, b_ref, o_ref):
    o_ref[...] = lax.dot_general(
        a_ref[...], b_ref[...],
        (((0,), (0,)), ((), ())),
        preferred_element_type=jnp.float32,
    )


def _score_matmul(uT, iT):
    grid = (B // _BM,)
    return pl.pallas_call(
        _matmul_body,
        grid=grid,
        in_specs=[
            pl.BlockSpec((EMB, _BM), lambda i: (0, i)),
            pl.BlockSpec((EMB, B), lambda i: (0, 0)),
        ],
        out_specs=pl.BlockSpec((_BM, B), lambda i: (i, 0)),
        out_shape=jax.ShapeDtypeStruct((B, B), jnp.float32),
    )(uT, iT)


def kernel(users, pos_items, user_table, item_table):
    uT, iT = _sc_gather(users, pos_items, user_table.T, item_table.T)
    return _score_matmul(uT, iT)


# speedup vs baseline: 1.0154x; 1.0098x over previous
import functools

import jax
import jax.numpy as jnp
from jax import lax
from jax.experimental import pallas as pl
from jax.experimental.pallas import tpu as pltpu
from jax.experimental.pallas import tpu_sc as plsc

B = 4096
EMB = 32
LANES = 128
_QC = 4


def _make_sc_gather():
    info = plsc.get_sparse_core_info()
    nc, ns = info.num_cores, info.num_subcores
    nw = nc * ns
    b_per_w = B // nw
    rounds = b_per_w // 16

    mesh = plsc.VectorSubcoreMesh(core_axis_name="c", subcore_axis_name="s")

    @functools.partial(
        pl.kernel,
        mesh=mesh,
        compiler_params=pltpu.CompilerParams(needs_layout_passes=False),
        out_type=[
            jax.ShapeDtypeStruct((EMB, B), jnp.float32),
            jax.ShapeDtypeStruct((EMB, B), jnp.float32),
        ],
        scratch_types=[
            pltpu.VMEM((B,), jnp.int32),
            pltpu.VMEM((B,), jnp.int32),
            pltpu.VMEM((2 * _QC, EMB, LANES), jnp.float32),
            pltpu.VMEM((2 * _QC, EMB, LANES), jnp.float32),
            pltpu.VMEM((EMB, b_per_w), jnp.float32),
            pltpu.VMEM((EMB, b_per_w), jnp.float32),
            pltpu.SemaphoreType.DMA,
            pltpu.SemaphoreType.DMA,
            pltpu.SemaphoreType.DMA,
            pltpu.SemaphoreType.DMA,
        ],
    )
    def sc_gather(users_hbm, items_hbm, utabT_hbm, itabT_hbm,
                  uoutT_hbm, ioutT_hbm,
                  uidx_all, iidx_all, ublk, iblk, urows, irows,
                  usem0, usem1, isem0, isem1):
        usems, isems = (usem0, usem1), (isem0, isem1)
        wid = lax.axis_index("s") * nc + lax.axis_index("c")
        base = pl.multiple_of(wid * b_per_w, b_per_w)
        pltpu.sync_copy(users_hbm, uidx_all)
        pltpu.sync_copy(items_hbm, iidx_all)

        row16 = lax.iota(jnp.int32, 16)

        nq = 16 // _QC

        def round_body(r, _):
            start = pl.multiple_of(base + r * 16, 16)
            uvec = uidx_all[pl.ds(start, 16)]
            ivec = iidx_all[pl.ds(start, 16)]

            def fire(q):
                slot = (q % 2) * _QC
                copies = []
                for j in range(_QC):
                    u = uvec[q * _QC + j]
                    uoff = pl.multiple_of((u // LANES) * LANES, LANES)
                    copies.append(pltpu.async_copy(
                        utabT_hbm.at[:, pl.ds(uoff, LANES)],
                        ublk.at[slot + j], usems[q % 2]))
                    i = ivec[q * _QC + j]
                    ioff = pl.multiple_of((i // LANES) * LANES, LANES)
                    copies.append(pltpu.async_copy(
                        itabT_hbm.at[:, pl.ds(ioff, LANES)],
                        iblk.at[slot + j], isems[q % 2]))
                return copies

            def extract(q):
                slot = (q % 2) * _QC
                for j in range(_QC):
                    b16 = jnp.full((16,), r * 16 + q * _QC + j, dtype=jnp.int32)
                    j16 = jnp.full((16,), slot + j, dtype=jnp.int32)
                    for vec, blk, rows in ((uvec, ublk, urows),
                                           (ivec, iblk, irows)):
                        lane16 = jnp.full(
                            (16,), vec[q * _QC + j] % LANES, dtype=jnp.int32)
                        lo = plsc.load_gather(blk, [j16, row16, lane16])
                        hi = plsc.load_gather(blk, [j16, row16 + 16, lane16])
                        plsc.store_scatter(rows, [row16, b16], lo)
                        plsc.store_scatter(rows, [row16 + 16, b16], hi)

            inflight = fire(0)
            for q in range(nq):
                nxt = fire(q + 1) if q + 1 < nq else []
                for c in inflight:
                    c.wait()
                extract(q)
                inflight = nxt
            return ()

        lax.fori_loop(0, rounds, round_body, ())

        pltpu.sync_copy(urows, uoutT_hbm.at[:, pl.ds(base, b_per_w)])
        pltpu.sync_copy(irows, ioutT_hbm.at[:, pl.ds(base, b_per_w)])

    return sc_gather


_sc_gather = _make_sc_gather()


_BM = 512


def _matmul_body(a_ref, b_ref, o_ref):
    o_ref[...] = lax.dot_general(
        a_ref[...], b_ref[...],
        (((0,), (0,)), ((), ())),
        preferred_element_type=jnp.float32,
    )


def _score_matmul(uT, iT):
    grid = (B // _BM,)
    return pl.pallas_call(
        _matmul_body,
        grid=grid,
        in_specs=[
            pl.BlockSpec((EMB, _BM), lambda i: (0, i)),
            pl.BlockSpec((EMB, B), lambda i: (0, 0)),
        ],
        out_specs=pl.BlockSpec((_BM, B), lambda i: (i, 0)),
        out_shape=jax.ShapeDtypeStruct((B, B), jnp.float32),
    )(uT, iT)


def kernel(users, pos_items, user_table, item_table):
    uT, iT = _sc_gather(users, pos_items, user_table.T, item_table.T)
    return _score_matmul(uT, iT)


# speedup vs baseline: 1.0332x; 1.0176x over previous
import functools

import jax
import jax.numpy as jnp
from jax import lax
from jax.experimental import pallas as pl
from jax.experimental.pallas import tpu as pltpu
from jax.experimental.pallas import tpu_sc as plsc

B = 4096
EMB = 32
LANES = 128
_QC = 4


def _make_sc_gather():
    info = plsc.get_sparse_core_info()
    nc, ns = info.num_cores, info.num_subcores
    nw = nc * ns
    b_per_w = B // nw
    rounds = b_per_w // 16

    mesh = plsc.VectorSubcoreMesh(core_axis_name="c", subcore_axis_name="s")

    @functools.partial(
        pl.kernel,
        mesh=mesh,
        compiler_params=pltpu.CompilerParams(needs_layout_passes=False),
        out_type=[
            jax.ShapeDtypeStruct((EMB, B), jnp.float32),
            jax.ShapeDtypeStruct((EMB, B), jnp.float32),
        ],
        scratch_types=[
            pltpu.VMEM((B,), jnp.int32),
            pltpu.VMEM((B,), jnp.int32),
            pltpu.VMEM((3 * _QC, EMB, LANES), jnp.float32),
            pltpu.VMEM((3 * _QC, EMB, LANES), jnp.float32),
            pltpu.VMEM((EMB, b_per_w), jnp.float32),
            pltpu.VMEM((EMB, b_per_w), jnp.float32),
            pltpu.SemaphoreType.DMA,
            pltpu.SemaphoreType.DMA,
            pltpu.SemaphoreType.DMA,
            pltpu.SemaphoreType.DMA,
            pltpu.SemaphoreType.DMA,
            pltpu.SemaphoreType.DMA,
        ],
    )
    def sc_gather(users_hbm, items_hbm, utabT_hbm, itabT_hbm,
                  uoutT_hbm, ioutT_hbm,
                  uidx_all, iidx_all, ublk, iblk, urows, irows,
                  usem0, usem1, usem2, isem0, isem1, isem2):
        usems, isems = (usem0, usem1, usem2), (isem0, isem1, isem2)
        wid = lax.axis_index("s") * nc + lax.axis_index("c")
        base = pl.multiple_of(wid * b_per_w, b_per_w)
        pltpu.sync_copy(users_hbm, uidx_all)
        pltpu.sync_copy(items_hbm, iidx_all)

        row16 = lax.iota(jnp.int32, 16)

        nq = 16 // _QC

        def round_body(r, _):
            start = pl.multiple_of(base + r * 16, 16)
            uvec = uidx_all[pl.ds(start, 16)]
            ivec = iidx_all[pl.ds(start, 16)]

            def fire(q):
                slot = (q % 3) * _QC
                copies = []
                for j in range(_QC):
                    u = uvec[q * _QC + j]
                    uoff = pl.multiple_of((u // LANES) * LANES, LANES)
                    copies.append(pltpu.async_copy(
                        utabT_hbm.at[:, pl.ds(uoff, LANES)],
                        ublk.at[slot + j], usems[q % 3]))
                    i = ivec[q * _QC + j]
                    ioff = pl.multiple_of((i // LANES) * LANES, LANES)
                    copies.append(pltpu.async_copy(
                        itabT_hbm.at[:, pl.ds(ioff, LANES)],
                        iblk.at[slot + j], isems[q % 3]))
                return copies

            def extract(q):
                slot = (q % 3) * _QC
                for j in range(_QC):
                    b16 = jnp.full((16,), r * 16 + q * _QC + j, dtype=jnp.int32)
                    j16 = jnp.full((16,), slot + j, dtype=jnp.int32)
                    for vec, blk, rows in ((uvec, ublk, urows),
                                           (ivec, iblk, irows)):
                        lane16 = jnp.full(
                            (16,), vec[q * _QC + j] % LANES, dtype=jnp.int32)
                        lo = plsc.load_gather(blk, [j16, row16, lane16])
                        hi = plsc.load_gather(blk, [j16, row16 + 16, lane16])
                        plsc.store_scatter(rows, [row16, b16], lo)
                        plsc.store_scatter(rows, [row16 + 16, b16], hi)

            pending = [fire(0), fire(1)]
            for q in range(nq):
                if q + 2 < nq:
                    pending.append(fire(q + 2))
                for c in pending.pop(0):
                    c.wait()
                extract(q)
            return ()

        lax.fori_loop(0, rounds, round_body, ())

        pltpu.sync_copy(urows, uoutT_hbm.at[:, pl.ds(base, b_per_w)])
        pltpu.sync_copy(irows, ioutT_hbm.at[:, pl.ds(base, b_per_w)])

    return sc_gather


_sc_gather = _make_sc_gather()


_BM = 512


def _matmul_body(a_ref, b_ref, o_ref):
    o_ref[...] = lax.dot_general(
        a_ref[...], b_ref[...],
        (((0,), (0,)), ((), ())),
        preferred_element_type=jnp.float32,
    )


def _score_matmul(uT, iT):
    grid = (B // _BM,)
    return pl.pallas_call(
        _matmul_body,
        grid=grid,
        in_specs=[
            pl.BlockSpec((EMB, _BM), lambda i: (0, i)),
            pl.BlockSpec((EMB, B), lambda i: (0, 0)),
        ],
        out_specs=pl.BlockSpec((_BM, B), lambda i: (i, 0)),
        out_shape=jax.ShapeDtypeStruct((B, B), jnp.float32),
    )(uT, iT)


def kernel(users, pos_items, user_table, item_table):
    uT, iT = _sc_gather(users, pos_items, user_table.T, item_table.T)
    return _score_matmul(uT, iT)


# speedup vs baseline: 1.0756x; 1.0410x over previous
import functools

import jax
import jax.numpy as jnp
from jax import lax
from jax.experimental import pallas as pl
from jax.experimental.pallas import tpu as pltpu
from jax.experimental.pallas import tpu_sc as plsc

B = 4096
EMB = 32
LANES = 128
_QC = 4


def _make_sc_gather():
    info = plsc.get_sparse_core_info()
    nc, ns = info.num_cores, info.num_subcores
    nw = nc * ns
    b_per_w = B // nw
    rounds = b_per_w // 16

    mesh = plsc.VectorSubcoreMesh(core_axis_name="c", subcore_axis_name="s")

    @functools.partial(
        pl.kernel,
        mesh=mesh,
        compiler_params=pltpu.CompilerParams(needs_layout_passes=False),
        out_type=[
            jax.ShapeDtypeStruct((EMB, B), jnp.float32),
            jax.ShapeDtypeStruct((EMB, B), jnp.float32),
        ],
        scratch_types=[
            pltpu.VMEM((B,), jnp.int32),
            pltpu.VMEM((B,), jnp.int32),
            pltpu.VMEM((3 * _QC, EMB, LANES), jnp.float32),
            pltpu.VMEM((3 * _QC, EMB, LANES), jnp.float32),
            pltpu.VMEM((EMB, b_per_w), jnp.float32),
            pltpu.VMEM((EMB, b_per_w), jnp.float32),
            pltpu.SemaphoreType.DMA,
            pltpu.SemaphoreType.DMA,
            pltpu.SemaphoreType.DMA,
            pltpu.SemaphoreType.DMA,
            pltpu.SemaphoreType.DMA,
            pltpu.SemaphoreType.DMA,
        ],
    )
    def sc_gather(users_hbm, items_hbm, utabT_hbm, itabT_hbm,
                  uoutT_hbm, ioutT_hbm,
                  uidx_all, iidx_all, ublk, iblk, urows, irows,
                  usem0, usem1, usem2, isem0, isem1, isem2):
        usems, isems = (usem0, usem1, usem2), (isem0, isem1, isem2)
        wid = lax.axis_index("s") * nc + lax.axis_index("c")
        base = pl.multiple_of(wid * b_per_w, b_per_w)
        pltpu.sync_copy(users_hbm, uidx_all)
        pltpu.sync_copy(items_hbm, iidx_all)

        row16 = lax.iota(jnp.int32, 16)

        nq = 16 // _QC

        def round_body(r, _):
            start0 = pl.multiple_of(base + r * 32, 16)
            start1 = pl.multiple_of(base + r * 32 + 16, 16)
            vecs = ((uidx_all[pl.ds(start0, 16)], iidx_all[pl.ds(start0, 16)]),
                    (uidx_all[pl.ds(start1, 16)], iidx_all[pl.ds(start1, 16)]))

            def fire(g):
                half, q = divmod(g, nq)
                uvec, ivec = vecs[half]
                slot = (g % 3) * _QC
                copies = []
                for j in range(_QC):
                    u = uvec[q * _QC + j]
                    uoff = pl.multiple_of((u // LANES) * LANES, LANES)
                    copies.append(pltpu.async_copy(
                        utabT_hbm.at[:, pl.ds(uoff, LANES)],
                        ublk.at[slot + j], usems[g % 3]))
                    i = ivec[q * _QC + j]
                    ioff = pl.multiple_of((i // LANES) * LANES, LANES)
                    copies.append(pltpu.async_copy(
                        itabT_hbm.at[:, pl.ds(ioff, LANES)],
                        iblk.at[slot + j], isems[g % 3]))
                return copies

            def extract(g):
                half, q = divmod(g, nq)
                uvec, ivec = vecs[half]
                slot = (g % 3) * _QC
                for j in range(_QC):
                    b16 = jnp.full((16,), r * 32 + g * _QC + j, dtype=jnp.int32)
                    j16 = jnp.full((16,), slot + j, dtype=jnp.int32)
                    for vec, blk, rows in ((uvec, ublk, urows),
                                           (ivec, iblk, irows)):
                        lane16 = jnp.full(
                            (16,), vec[q * _QC + j] % LANES, dtype=jnp.int32)
                        lo = plsc.load_gather(blk, [j16, row16, lane16])
                        hi = plsc.load_gather(blk, [j16, row16 + 16, lane16])
                        plsc.store_scatter(rows, [row16, b16], lo)
                        plsc.store_scatter(rows, [row16 + 16, b16], hi)

            pending = [fire(0), fire(1)]
            for g in range(2 * nq):
                if g + 2 < 2 * nq:
                    pending.append(fire(g + 2))
                for c in pending.pop(0):
                    c.wait()
                extract(g)
            return ()

        lax.fori_loop(0, rounds // 2, round_body, ())

        pltpu.sync_copy(urows, uoutT_hbm.at[:, pl.ds(base, b_per_w)])
        pltpu.sync_copy(irows, ioutT_hbm.at[:, pl.ds(base, b_per_w)])

    return sc_gather


_sc_gather = _make_sc_gather()


_BM = 512


def _matmul_body(a_ref, b_ref, o_ref):
    o_ref[...] = lax.dot_general(
        a_ref[...], b_ref[...],
        (((0,), (0,)), ((), ())),
        preferred_element_type=jnp.float32,
    )


def _score_matmul(uT, iT):
    grid = (B // _BM,)
    return pl.pallas_call(
        _matmul_body,
        grid=grid,
        in_specs=[
            pl.BlockSpec((EMB, _BM), lambda i: (0, i)),
            pl.BlockSpec((EMB, B), lambda i: (0, 0)),
        ],
        out_specs=pl.BlockSpec((_BM, B), lambda i: (i, 0)),
        out_shape=jax.ShapeDtypeStruct((B, B), jnp.float32),
    )(uT, iT)


def kernel(users, pos_items, user_table, item_table):
    uT, iT = _sc_gather(users, pos_items, user_table.T, item_table.T)
    return _score_matmul(uT, iT)
